# baseline (speedup 1.0000x reference)
import functools

import jax
import jax.numpy as jnp
from jax import lax
from jax.experimental import pallas as pl
from jax.experimental.pallas import tpu as pltpu
from jax.experimental.pallas import tpu_sc as plsc

NC, NS, LANES = 2, 16, 16
TILES = NC * NS
CHUNK = 128
DP = 128

N = 10000
E = 320000
ET = E + N
CPT = -(-ET // (TILES * CHUNK))
CPT += CPT % 2
ETP = TILES * CHUNK * CPT
ETP2 = ETP + 2 * CHUNK
EPT = CPT * CHUNK
RPT = -(-(-(-N // NS)) // CHUNK) * CHUNK
NP_ = RPT * NS


def _mesh():
    return plsc.VectorSubcoreMesh(
        core_axis_name="c", subcore_axis_name="s", num_cores=NC, num_subcores=NS
    )


def _zero_1d(buf, n):
    z = jnp.zeros((LANES,), jnp.float32)
    for j in range(n // LANES):
        buf[pl.ds(j * LANES, LANES)] = z


def _zero_2d(buf, n, d):
    z = jnp.zeros((LANES,), jnp.float32)

    def body(j, carry):
        for kk in range(d // LANES):
            buf[j, pl.ds(kk * LANES, LANES)] = z
        return carry

    lax.fori_loop(0, n, body, 0)


def _deg_body(col_hbm, w_hbm, deg_hbm, col_res, w_res, zbuf, acc, rc, rw, ss):
    cid = lax.axis_index("c")
    sid = lax.axis_index("s")
    wid = cid * NS + sid
    start = sid * RPT
    pltpu.async_copy(col_hbm.at[pl.ds(wid * EPT, EPT)], col_res, rc)
    pltpu.async_copy(w_hbm.at[pl.ds(wid * EPT, EPT)], w_res, rw)
    _zero_1d(zbuf, CHUNK)
    for t in range(RPT // CHUNK):
        pltpu.sync_copy(zbuf, acc.at[pl.ds(start + t * CHUNK, CHUNK)])
    plsc.subcore_barrier()
    pltpu.make_async_copy(col_hbm.at[pl.ds(wid * EPT, EPT)], col_res, rc).wait()
    pltpu.make_async_copy(w_hbm.at[pl.ds(wid * EPT, EPT)], w_res, rw).wait()

    def body(ch, carry):
        sl = pl.ds(ch * CHUNK, CHUNK)
        pltpu.async_copy(w_res.at[sl], acc.at[col_res.at[sl]], ss, add=True)
        pltpu.make_async_copy(w_res.at[sl], acc.at[col_res.at[sl]], ss).wait()
        return carry

    lax.fori_loop(0, CPT, body, 0)
    plsc.subcore_barrier()
    for t in range(RPT // CHUNK):
        pltpu.sync_copy(acc.at[pl.ds(start + t * CHUNK, CHUNK)],
                        deg_hbm.at[pl.ds(cid * NP_ + start + t * CHUNK,
                                         CHUNK)])


def _sc_degree(col, w):
    k = pl.kernel(
        _deg_body,
        out_type=jax.ShapeDtypeStruct((NC * NP_,), jnp.float32),
        mesh=_mesh(),
        scratch_types=[
            pltpu.VMEM((EPT,), jnp.int32),
            pltpu.VMEM((EPT,), jnp.float32),
            pltpu.VMEM((CHUNK,), jnp.float32),
            pltpu.VMEM_SHARED((NP_,), jnp.float32),
            pltpu.SemaphoreType.DMA,
            pltpu.SemaphoreType.DMA,
            pltpu.SemaphoreType.DMA,
        ],
    )
    return k(col, w)


def _seg_body(m_hbm, col_hbm, out_hbm, col_res, cmp, acc, rc, rm, ss):
    DR = DP
    cid = lax.axis_index("c")
    sid = lax.axis_index("s")
    wid = cid * NS + sid
    start = sid * RPT

    pltpu.async_copy(col_hbm.at[pl.ds(wid * EPT, EPT)], col_res, rc)
    _zero_2d(cmp, CHUNK, DR)
    for t in range(RPT // CHUNK):
        pltpu.sync_copy(cmp, acc.at[pl.ds(start + t * CHUNK, CHUNK)])
    plsc.subcore_barrier()
    pltpu.make_async_copy(col_hbm.at[pl.ds(wid * EPT, EPT)], col_res, rc).wait()

    def body(ch, carry):
        esl = pl.ds(ch * CHUNK, CHUNK)
        base = wid * EPT + ch * CHUNK
        pltpu.async_copy(m_hbm.at[pl.ds(base, CHUNK)], cmp, rm)
        pltpu.make_async_copy(m_hbm.at[pl.ds(base, CHUNK)], cmp, rm).wait()
        pltpu.async_copy(cmp, acc.at[col_res.at[esl]], ss, add=True)
        pltpu.make_async_copy(cmp, acc.at[col_res.at[esl]], ss).wait()
        return carry

    lax.fori_loop(0, CPT, body, 0)
    plsc.subcore_barrier()
    for t in range(RPT // CHUNK):
        pltpu.sync_copy(acc.at[pl.ds(start + t * CHUNK, CHUNK)],
                        out_hbm.at[pl.ds(cid * NP_ + start + t * CHUNK,
                                         CHUNK)])


def _sc_segsum(m, col):
    k = pl.kernel(
        _seg_body,
        out_type=jax.ShapeDtypeStruct((NC * NP_, DP), jnp.float32),
        mesh=_mesh(),
        scratch_types=[
            pltpu.VMEM((EPT,), jnp.int32),
            pltpu.VMEM((CHUNK, DP), jnp.float32),
            pltpu.VMEM_SHARED((NP_, DP), jnp.float32),
            pltpu.SemaphoreType.DMA,
            pltpu.SemaphoreType.DMA,
            pltpu.SemaphoreType.DMA,
        ],
    )
    return k(m, col)


def _tc1_body(dpt_ref, x_ref, w1_ref, dinv_ref, g1_ref):
    deg = dpt_ref[:, 0:1] + dpt_ref[:, 1:2]
    dinv = jnp.where(deg > 0, lax.rsqrt(deg), 0.0)
    dinv_ref[...] = dinv
    h = lax.dot_general(
        x_ref[...], w1_ref[...], (((1,), (0,)), ((), ())),
        precision=lax.Precision.HIGHEST, preferred_element_type=jnp.float32)
    g1_ref[:, 0:64] = dinv[:N] * h
    g1_ref[:, 64:DP] = jnp.zeros((N, DP - 64), jnp.float32)


def _tc_mid_body(DI, DO, p_ref, dinv_ref, b_ref, w_ref, g_ref):
    dinv = dinv_ref[...]
    p = p_ref[0, :, 0:DI] + p_ref[1, :, 0:DI]
    h = jnp.maximum(dinv * p + b_ref[...], 0.0)
    g_ref[:, 0:DO] = dinv * lax.dot_general(
        h, w_ref[...], (((1,), (0,)), ((), ())),
        precision=lax.Precision.HIGHEST, preferred_element_type=jnp.float32)
    g_ref[:, DO:DP] = jnp.zeros((NP_, DP - DO), jnp.float32)


def _tc_fin_body(p_ref, dinv_ref, b_ref, node_ref, graph_ref):
    p = p_ref[0, :N, 0:32] + p_ref[1, :N, 0:32]
    h = dinv_ref[:N] * p + b_ref[...]
    node_ref[...] = h
    graph_ref[...] = jnp.mean(h, axis=0, keepdims=True)


def kernel(x, edge_index, edge_attr, W1, b1, W2, b2, W3, b3):
    f32 = jnp.float32
    loop = jnp.arange(N, dtype=edge_index.dtype)
    pad = ETP2 - ET
    pad_idx = (jnp.arange(pad) % N).astype(edge_index.dtype)
    rowr = jnp.concatenate([edge_index[0], loop])
    col = jnp.concatenate([edge_index[1], loop, pad_idx])
    wfull = jnp.concatenate([edge_attr.reshape(-1), jnp.ones((N,), f32)])
    mpad = jnp.zeros((ETP - ET, DP), f32)

    degp = _sc_degree(col, jnp.concatenate([wfull, jnp.zeros((pad,), f32)]))
    dpt = degp.reshape(NC, NP_).T

    dinv, g1 = pl.pallas_call(
        _tc1_body,
        out_shape=[jax.ShapeDtypeStruct((NP_, 1), f32),
                   jax.ShapeDtypeStruct((N, DP), f32)],
    )(dpt, x, W1)

    def msgs(gtab):
        m = wfull[:, None] * jnp.take(gtab, rowr, axis=0)
        return jnp.concatenate([m, mpad])

    p1 = _sc_segsum(msgs(g1), col)

    g2 = pl.pallas_call(
        functools.partial(_tc_mid_body, 64, 64),
        out_shape=jax.ShapeDtypeStruct((NP_, DP), f32),
    )(p1.reshape(NC, NP_, DP), dinv, b1.reshape(1, 64), W2)

    p2 = _sc_segsum(msgs(g2), col)

    g3 = pl.pallas_call(
        functools.partial(_tc_mid_body, 64, 32),
        out_shape=jax.ShapeDtypeStruct((NP_, DP), f32),
    )(p2.reshape(NC, NP_, DP), dinv, b2.reshape(1, 64), W3)

    p3 = _sc_segsum(msgs(g3), col)

    node, graph = pl.pallas_call(
        _tc_fin_body,
        out_shape=[jax.ShapeDtypeStruct((N, 32), f32),
                   jax.ShapeDtypeStruct((1, 32), f32)],
    )(p3.reshape(NC, NP_, DP), dinv, b3.reshape(1, 32))

    return (node, graph)


# speedup vs baseline: 1.0629x; 1.0629x over previous
import functools

import jax
import jax.numpy as jnp
from jax import lax
from jax.experimental import pallas as pl
from jax.experimental.pallas import tpu as pltpu
from jax.experimental.pallas import tpu_sc as plsc

NC, NS, LANES = 2, 16, 16
TILES = NC * NS
CHUNK = 128
DP = 128

N = 10000
E = 320000
ET = E + N
CPT = -(-ET // (TILES * CHUNK))
CPT += CPT % 2
ETP = TILES * CHUNK * CPT
ETP2 = ETP + 2 * CHUNK
EPT = CPT * CHUNK
RPT = -(-(-(-N // NS)) // CHUNK) * CHUNK
NP_ = RPT * NS


def _mesh():
    return plsc.VectorSubcoreMesh(
        core_axis_name="c", subcore_axis_name="s", num_cores=NC, num_subcores=NS
    )


def _zero_1d(buf, n):
    z = jnp.zeros((LANES,), jnp.float32)
    for j in range(n // LANES):
        buf[pl.ds(j * LANES, LANES)] = z


def _zero_2d(buf, n, d):
    z = jnp.zeros((LANES,), jnp.float32)

    def body(j, carry):
        for kk in range(d // LANES):
            buf[j, pl.ds(kk * LANES, LANES)] = z
        return carry

    lax.fori_loop(0, n, body, 0)


def _deg_body(col_hbm, w_hbm, deg_hbm, col_res, w_res, zbuf, acc, rc, rw, ss):
    cid = lax.axis_index("c")
    sid = lax.axis_index("s")
    wid = cid * NS + sid
    start = sid * RPT
    pltpu.async_copy(col_hbm.at[pl.ds(wid * EPT, EPT)], col_res, rc)
    pltpu.async_copy(w_hbm.at[pl.ds(wid * EPT, EPT)], w_res, rw)
    _zero_1d(zbuf, CHUNK)
    for t in range(RPT // CHUNK):
        pltpu.sync_copy(zbuf, acc.at[pl.ds(start + t * CHUNK, CHUNK)])
    plsc.subcore_barrier()
    pltpu.make_async_copy(col_hbm.at[pl.ds(wid * EPT, EPT)], col_res, rc).wait()
    pltpu.make_async_copy(w_hbm.at[pl.ds(wid * EPT, EPT)], w_res, rw).wait()

    def body(ch, carry):
        sl = pl.ds(ch * CHUNK, CHUNK)
        pltpu.async_copy(w_res.at[sl], acc.at[col_res.at[sl]], ss, add=True)
        pltpu.make_async_copy(w_res.at[sl], acc.at[col_res.at[sl]], ss).wait()
        return carry

    lax.fori_loop(0, CPT, body, 0)
    plsc.subcore_barrier()
    for t in range(RPT // CHUNK):
        pltpu.sync_copy(acc.at[pl.ds(start + t * CHUNK, CHUNK)],
                        deg_hbm.at[pl.ds(cid * NP_ + start + t * CHUNK,
                                         CHUNK)])


def _sc_degree(col, w):
    k = pl.kernel(
        _deg_body,
        out_type=jax.ShapeDtypeStruct((NC * NP_,), jnp.float32),
        mesh=_mesh(),
        scratch_types=[
            pltpu.VMEM((EPT,), jnp.int32),
            pltpu.VMEM((EPT,), jnp.float32),
            pltpu.VMEM((CHUNK,), jnp.float32),
            pltpu.VMEM_SHARED((NP_,), jnp.float32),
            pltpu.SemaphoreType.DMA,
            pltpu.SemaphoreType.DMA,
            pltpu.SemaphoreType.DMA,
        ],
    )
    return k(col, w)


def _seg_body(m_hbm, col_hbm, out_hbm, col_res, cmpA, cmpB, acc,
              rc, rmA, rmB, ss):
    DR = DP
    cid = lax.axis_index("c")
    sid = lax.axis_index("s")
    wid = cid * NS + sid
    start = sid * RPT

    def m_start(ch, cmp, sem):
        base = wid * EPT + ch * CHUNK
        pltpu.async_copy(m_hbm.at[pl.ds(base, CHUNK)], cmp, sem)

    def m_wait(ch, cmp, sem):
        base = wid * EPT + ch * CHUNK
        pltpu.make_async_copy(m_hbm.at[pl.ds(base, CHUNK)], cmp, sem).wait()

    def scat(ch, cmp):
        esl = pl.ds(ch * CHUNK, CHUNK)
        pltpu.async_copy(cmp, acc.at[col_res.at[esl]], ss, add=True)
        pltpu.make_async_copy(cmp, acc.at[col_res.at[esl]], ss).wait()

    pltpu.async_copy(col_hbm.at[pl.ds(wid * EPT, EPT)], col_res, rc)
    _zero_2d(cmpA, CHUNK, DR)
    for t in range(RPT // CHUNK):
        pltpu.sync_copy(cmpA, acc.at[pl.ds(start + t * CHUNK, CHUNK)])
    plsc.subcore_barrier()
    pltpu.make_async_copy(col_hbm.at[pl.ds(wid * EPT, EPT)], col_res, rc).wait()

    m_start(0, cmpA, rmA)
    m_start(1, cmpB, rmB)

    def body(k, carry):
        a = 2 * k
        m_wait(a, cmpA, rmA)
        scat(a, cmpA)
        m_start(a + 2, cmpA, rmA)
        m_wait(a + 1, cmpB, rmB)
        scat(a + 1, cmpB)
        m_start(a + 3, cmpB, rmB)
        return carry

    lax.fori_loop(0, CPT // 2, body, 0)
    m_wait(CPT, cmpA, rmA)
    m_wait(CPT + 1, cmpB, rmB)
    plsc.subcore_barrier()
    for t in range(RPT // CHUNK):
        pltpu.sync_copy(acc.at[pl.ds(start + t * CHUNK, CHUNK)],
                        out_hbm.at[pl.ds(cid * NP_ + start + t * CHUNK,
                                         CHUNK)])


def _sc_segsum(m, col):
    k = pl.kernel(
        _seg_body,
        out_type=jax.ShapeDtypeStruct((NC * NP_, DP), jnp.float32),
        mesh=_mesh(),
        scratch_types=[
            pltpu.VMEM((EPT,), jnp.int32),
            pltpu.VMEM((CHUNK, DP), jnp.float32),
            pltpu.VMEM((CHUNK, DP), jnp.float32),
            pltpu.VMEM_SHARED((NP_, DP), jnp.float32),
            pltpu.SemaphoreType.DMA,
            pltpu.SemaphoreType.DMA,
            pltpu.SemaphoreType.DMA,
            pltpu.SemaphoreType.DMA,
        ],
    )
    return k(m, col)


def _tc1_body(dpt_ref, x_ref, w1_ref, dinv_ref, g1_ref):
    deg = dpt_ref[:, 0:1] + dpt_ref[:, 1:2]
    dinv = jnp.where(deg > 0, lax.rsqrt(deg), 0.0)
    dinv_ref[...] = dinv
    h = lax.dot_general(
        x_ref[...], w1_ref[...], (((1,), (0,)), ((), ())),
        precision=lax.Precision.HIGHEST, preferred_element_type=jnp.float32)
    g1_ref[:, 0:64] = dinv[:N] * h
    g1_ref[:, 64:DP] = jnp.zeros((N, DP - 64), jnp.float32)


def _tc_mid_body(DI, DO, p_ref, dinv_ref, b_ref, w_ref, g_ref):
    dinv = dinv_ref[...]
    p = p_ref[0, :, 0:DI] + p_ref[1, :, 0:DI]
    h = jnp.maximum(dinv * p + b_ref[...], 0.0)
    g_ref[:, 0:DO] = dinv * lax.dot_general(
        h, w_ref[...], (((1,), (0,)), ((), ())),
        precision=lax.Precision.HIGHEST, preferred_element_type=jnp.float32)
    g_ref[:, DO:DP] = jnp.zeros((NP_, DP - DO), jnp.float32)


def _tc_fin_body(p_ref, dinv_ref, b_ref, node_ref, graph_ref):
    p = p_ref[0, :N, 0:32] + p_ref[1, :N, 0:32]
    h = dinv_ref[:N] * p + b_ref[...]
    node_ref[...] = h
    graph_ref[...] = jnp.mean(h, axis=0, keepdims=True)


def kernel(x, edge_index, edge_attr, W1, b1, W2, b2, W3, b3):
    f32 = jnp.float32
    loop = jnp.arange(N, dtype=edge_index.dtype)
    pad = ETP2 - ET
    pad_idx = (jnp.arange(pad) % N).astype(edge_index.dtype)
    rowr = jnp.concatenate([edge_index[0], loop])
    col = jnp.concatenate([edge_index[1], loop, pad_idx])
    wfull = jnp.concatenate([edge_attr.reshape(-1), jnp.ones((N,), f32)])
    mpad = jnp.zeros((ETP2 - ET, DP), f32)

    degp = _sc_degree(col, jnp.concatenate([wfull, jnp.zeros((pad,), f32)]))
    dpt = degp.reshape(NC, NP_).T

    dinv, g1 = pl.pallas_call(
        _tc1_body,
        out_shape=[jax.ShapeDtypeStruct((NP_, 1), f32),
                   jax.ShapeDtypeStruct((N, DP), f32)],
    )(dpt, x, W1)

    def msgs(gtab):
        m = wfull[:, None] * jnp.take(gtab, rowr, axis=0)
        return jnp.concatenate([m, mpad])

    p1 = _sc_segsum(msgs(g1), col)

    g2 = pl.pallas_call(
        functools.partial(_tc_mid_body, 64, 64),
        out_shape=jax.ShapeDtypeStruct((NP_, DP), f32),
    )(p1.reshape(NC, NP_, DP), dinv, b1.reshape(1, 64), W2)

    p2 = _sc_segsum(msgs(g2), col)

    g3 = pl.pallas_call(
        functools.partial(_tc_mid_body, 64, 32),
        out_shape=jax.ShapeDtypeStruct((NP_, DP), f32),
    )(p2.reshape(NC, NP_, DP), dinv, b2.reshape(1, 64), W3)

    p3 = _sc_segsum(msgs(g3), col)

    node, graph = pl.pallas_call(
        _tc_fin_body,
        out_shape=[jax.ShapeDtypeStruct((N, 32), f32),
                   jax.ShapeDtypeStruct((1, 32), f32)],
    )(p3.reshape(NC, NP_, DP), dinv, b3.reshape(1, 32))

    return (node, graph)
